# Initial kernel scaffold; baseline (speedup 1.0000x reference)
import jax
import jax.numpy as jnp
from jax.experimental import pallas as pl


def kernel(pc, mask):
    raise NotImplementedError("write your pallas kernel here")


# speedup vs baseline: 20.5651x; 20.5651x over previous
import functools

import jax
import jax.numpy as jnp
from jax import lax
from jax.experimental import pallas as pl
from jax.experimental.pallas import tpu as pltpu
from jax.experimental.pallas import tpu_sc as plsc

N = 16384
K_NN = 12
RADIUS2 = 1.0
C = 100
CPAD = 128
NCHUNK = CPAD // 16

ROWS = 128
GRID = N // ROWS

NW = 32
PPW = N // NW
BATCH = 32
NB = PPW // BATCH
IDXB = BATCH * K_NN


DPAD = 128


def _knn_body(q_ref, pcT_ref, sel_ref, d2_ref):
    q = q_ref[...]
    pcT = pcT_ref[...]
    qn = jnp.sum(q * q, axis=1, keepdims=True)
    kn = jnp.sum(pcT * pcT, axis=0, keepdims=True)
    dot = lax.dot_general(q, pcT, (((1,), (0,)), ((), ())),
                          preferred_element_type=jnp.float32,
                          precision=lax.Precision.DEFAULT)
    d2_ref[...] = qn - 2.0 * dot + kn

    iota = lax.broadcasted_iota(jnp.int32, (ROWS, N), 1)
    kiota = lax.broadcasted_iota(jnp.int32, (ROWS, K_NN), 1)
    big_i = jnp.int32(N)
    inf = jnp.float32(jnp.inf)

    def select_body(k, carry):
        vals, idx = carry
        d2 = d2_ref[...]
        m = jnp.min(d2, axis=1, keepdims=True)
        at_min = d2 == m
        sel_idx = jnp.min(jnp.where(at_min, iota, big_i),
                          axis=1, keepdims=True)
        d2_ref[...] = jnp.where(iota == sel_idx, inf, d2)
        vals = jnp.where(kiota == k, m, vals)
        idx = jnp.where(kiota == k, sel_idx, idx)
        return vals, idx

    vals0 = jnp.zeros((ROWS, K_NN), jnp.float32)
    idx0 = jnp.zeros((ROWS, K_NN), jnp.int32)
    vals, idx = lax.fori_loop(0, K_NN, select_body, (vals0, idx0))
    sel_ref[...] = jnp.where(vals > RADIUS2, idx[:, 0:1], idx)


def _knn_indices(pc2, pcT):
    return pl.pallas_call(
        _knn_body,
        grid=(GRID,),
        in_specs=[
            pl.BlockSpec((ROWS, DPAD), lambda i: (i, 0)),
            pl.BlockSpec((DPAD, N), lambda i: (0, 0)),
        ],
        out_specs=pl.BlockSpec((ROWS, K_NN), lambda i: (i, 0)),
        out_shape=jax.ShapeDtypeStruct((N, K_NN), jnp.int32),
        scratch_shapes=[pltpu.VMEM((ROWS, N), jnp.float32)],
    )(pc2, pcT)


def _lane_shuffle(x, perm2d):
    dnums = lax.GatherDimensionNumbers(
        offset_dims=(), collapsed_slice_dims=(0,), start_index_map=(0,))
    return lax.gather(x, perm2d, dnums, slice_sizes=(1,),
                      mode=lax.GatherScatterMode.PROMISE_IN_BOUNDS)


def _sc_loss_body(mask_hbm, selflat_hbm, out_hbm, idx_v, nb_v, self_v,
                  out_v, sem):
    wid = lax.axis_index("s") * 2 + lax.axis_index("c")

    def batch_body(b, carry):
        rbase = wid * PPW + b * BATCH
        goff = rbase * K_NN
        pltpu.sync_copy(selflat_hbm.at[pl.ds(goff, IDXB)], idx_v)
        copies = []
        for j in range(IDXB // 128):
            copies.append(
                pltpu.async_copy(
                    mask_hbm.at[idx_v.at[pl.ds(j * 128, 128)]],
                    nb_v.at[pl.ds(j * 128, 128)], sem))
        pltpu.sync_copy(mask_hbm.at[pl.ds(rbase, BATCH)], self_v)
        for cp in copies:
            cp.wait()

        lane = lax.iota(jnp.int32, 16)
        perms = [(lane ^ sh)[:, None] for sh in (8, 4, 2, 1)]

        def point_body(p, carry2):
            a = [self_v[p, pl.ds(c * 16, 16)] for c in range(NCHUNK)]
            row = jnp.zeros((16,), jnp.float32)
            for k in range(K_NN):
                r = p * K_NN + k
                acc = jnp.abs(a[0] - nb_v[r, pl.ds(0, 16)])
                for c in range(1, NCHUNK):
                    acc = acc + jnp.abs(a[c] - nb_v[r, pl.ds(c * 16, 16)])
                for perm in perms:
                    acc = acc + _lane_shuffle(acc, perm)
                row = jnp.where(lane == k, acc, row)
            out_v[p, :] = row
            return carry2

        lax.fori_loop(0, BATCH, point_body, 0)
        pltpu.sync_copy(out_v, out_hbm.at[pl.ds(rbase, BATCH)])
        return carry

    lax.fori_loop(0, NB, batch_body, 0)


@functools.cache
def _sc_loss_kernel():
    return pl.kernel(
        _sc_loss_body,
        mesh=plsc.VectorSubcoreMesh(core_axis_name="c", subcore_axis_name="s"),
        out_type=jax.ShapeDtypeStruct((N, 16), jnp.float32),
        scratch_types=[
            pltpu.VMEM((IDXB,), jnp.int32),
            pltpu.VMEM((IDXB, CPAD), jnp.float32),
            pltpu.VMEM((BATCH, CPAD), jnp.float32),
            pltpu.VMEM((BATCH, 16), jnp.float32),
            pltpu.SemaphoreType.DMA,
        ],
    )


def kernel(pc, mask):
    pc2 = jnp.pad(pc[0], ((0, 0), (0, DPAD - 3)))
    pcT = jnp.transpose(pc2)
    sel = _knn_indices(pc2, pcT)
    maskp = jnp.pad(mask[0], ((0, 0), (0, CPAD - C)))
    out16 = _sc_loss_kernel()(maskp, sel.reshape(-1))
    per_point = out16[:, :K_NN][None]
    smooth_loss = jnp.mean(per_point)
    return (smooth_loss, per_point)


# speedup vs baseline: 22.1000x; 1.0746x over previous
import functools

import jax
import jax.numpy as jnp
from jax import lax
from jax.experimental import pallas as pl
from jax.experimental.pallas import tpu as pltpu
from jax.experimental.pallas import tpu_sc as plsc

N = 16384
K_NN = 12
RADIUS2 = 1.0
C = 100
CPAD = 128
NCHUNK = CPAD // 16

ROWS = 256
GRID = N // ROWS

NW = 32
PPW = N // NW
BATCH = 32
NB = PPW // BATCH
IDXB = BATCH * K_NN


DPAD = 128


def _knn_body(q_ref, pcT_ref, sel_ref, d2_ref):
    q = q_ref[...]
    pcT = pcT_ref[...]
    qn = jnp.sum(q * q, axis=1, keepdims=True)
    kn = jnp.sum(pcT * pcT, axis=0, keepdims=True)
    dot = lax.dot_general(q, pcT, (((1,), (0,)), ((), ())),
                          preferred_element_type=jnp.float32,
                          precision=lax.Precision.DEFAULT)
    d2_ref[...] = qn - 2.0 * dot + kn

    iota = lax.broadcasted_iota(jnp.int32, (ROWS, N), 1)
    kiota = lax.broadcasted_iota(jnp.int32, (ROWS, K_NN), 1)
    big_i = jnp.int32(N)
    inf = jnp.float32(jnp.inf)

    def select_body(k, carry):
        vals, idx = carry
        d2 = d2_ref[...]
        m = jnp.min(d2, axis=1, keepdims=True)
        at_min = d2 == m
        sel_idx = jnp.min(jnp.where(at_min, iota, big_i),
                          axis=1, keepdims=True)
        d2_ref[...] = jnp.where(iota == sel_idx, inf, d2)
        vals = jnp.where(kiota == k, m, vals)
        idx = jnp.where(kiota == k, sel_idx, idx)
        return vals, idx

    vals0 = jnp.zeros((ROWS, K_NN), jnp.float32)
    idx0 = jnp.zeros((ROWS, K_NN), jnp.int32)
    vals, idx = lax.fori_loop(0, K_NN, select_body, (vals0, idx0))
    sel_ref[...] = jnp.where(vals > RADIUS2, idx[:, 0:1], idx)


def _knn_indices(pc2, pcT):
    return pl.pallas_call(
        _knn_body,
        grid=(GRID,),
        in_specs=[
            pl.BlockSpec((ROWS, DPAD), lambda i: (i, 0)),
            pl.BlockSpec((DPAD, N), lambda i: (0, 0)),
        ],
        out_specs=pl.BlockSpec((ROWS, K_NN), lambda i: (i, 0)),
        out_shape=jax.ShapeDtypeStruct((N, K_NN), jnp.int32),
        scratch_shapes=[pltpu.VMEM((ROWS, N), jnp.float32)],
    )(pc2, pcT)


def _lane_shuffle(x, perm2d):
    dnums = lax.GatherDimensionNumbers(
        offset_dims=(), collapsed_slice_dims=(0,), start_index_map=(0,))
    return lax.gather(x, perm2d, dnums, slice_sizes=(1,),
                      mode=lax.GatherScatterMode.PROMISE_IN_BOUNDS)


def _sc_loss_body(mask_hbm, selflat_hbm, out_hbm, idx_v, nb_v, self_v,
                  out_v, sem):
    wid = lax.axis_index("s") * 2 + lax.axis_index("c")

    def batch_body(b, carry):
        rbase = wid * PPW + b * BATCH
        goff = rbase * K_NN
        pltpu.sync_copy(selflat_hbm.at[pl.ds(goff, IDXB)], idx_v)
        copies = []
        for j in range(IDXB // 128):
            copies.append(
                pltpu.async_copy(
                    mask_hbm.at[idx_v.at[pl.ds(j * 128, 128)]],
                    nb_v.at[pl.ds(j * 128, 128)], sem))
        pltpu.sync_copy(mask_hbm.at[pl.ds(rbase, BATCH)], self_v)
        for cp in copies:
            cp.wait()

        lane = lax.iota(jnp.int32, 16)
        perms = [(lane ^ sh)[:, None] for sh in (8, 4, 2, 1)]

        def point_body(p, carry2):
            a = [self_v[p, pl.ds(c * 16, 16)] for c in range(NCHUNK)]
            row = jnp.zeros((16,), jnp.float32)
            for k in range(K_NN):
                r = p * K_NN + k
                acc = jnp.abs(a[0] - nb_v[r, pl.ds(0, 16)])
                for c in range(1, NCHUNK):
                    acc = acc + jnp.abs(a[c] - nb_v[r, pl.ds(c * 16, 16)])
                for perm in perms:
                    acc = acc + _lane_shuffle(acc, perm)
                row = jnp.where(lane == k, acc, row)
            out_v[p, :] = row
            return carry2

        lax.fori_loop(0, BATCH, point_body, 0)
        pltpu.sync_copy(out_v, out_hbm.at[pl.ds(rbase, BATCH)])
        return carry

    lax.fori_loop(0, NB, batch_body, 0)


@functools.cache
def _sc_loss_kernel():
    return pl.kernel(
        _sc_loss_body,
        mesh=plsc.VectorSubcoreMesh(core_axis_name="c", subcore_axis_name="s"),
        out_type=jax.ShapeDtypeStruct((N, 16), jnp.float32),
        scratch_types=[
            pltpu.VMEM((IDXB,), jnp.int32),
            pltpu.VMEM((IDXB, CPAD), jnp.float32),
            pltpu.VMEM((BATCH, CPAD), jnp.float32),
            pltpu.VMEM((BATCH, 16), jnp.float32),
            pltpu.SemaphoreType.DMA,
        ],
    )


def kernel(pc, mask):
    pc2 = jnp.pad(pc[0], ((0, 0), (0, DPAD - 3)))
    pcT = jnp.transpose(pc2)
    sel = _knn_indices(pc2, pcT)
    maskp = jnp.pad(mask[0], ((0, 0), (0, CPAD - C)))
    out16 = _sc_loss_kernel()(maskp, sel.reshape(-1))
    per_point = out16[:, :K_NN][None]
    smooth_loss = jnp.mean(per_point)
    return (smooth_loss, per_point)
